# baseline (speedup 1.0000x reference)
import functools

import jax
import jax.numpy as jnp
from jax import lax
from jax.experimental import pallas as pl
from jax.experimental.pallas import tpu as pltpu
from jax.experimental.pallas import tpu_sc as plsc

B = 16384
D = 64
LAT = 128
NC = 2
NS = 16
NW = NC * NS
BPW = B // NW
ICH = 128
NCH = BPW // ICH


def _body(uidx_h, iidx_h, uT_h, iT_h, uA_h, iA_h, w_h, b_h, out_h,
          uidx_v, iidx_v, urows, irows, uA_v, iA_v, w_v, b_v,
          wu_v, wi_v, out_v, gsem):
    cid = lax.axis_index("c")
    sid = lax.axis_index("s")
    wid = sid * NC + cid
    base = wid * BPW

    pltpu.sync_copy(uidx_h.at[wid], uidx_v)
    pltpu.sync_copy(iidx_h.at[wid], iidx_v)
    copies = []
    for j in range(NCH):
        copies.append(pltpu.async_copy(
            uT_h.at[uidx_v.at[j]], urows.at[pl.ds(j * ICH, ICH)], gsem))
        copies.append(pltpu.async_copy(
            iT_h.at[iidx_v.at[j]], irows.at[pl.ds(j * ICH, ICH)], gsem))

    pltpu.sync_copy(uA_h, uA_v)
    pltpu.sync_copy(iA_h, iA_v)
    pltpu.sync_copy(w_h, w_v)
    pltpu.sync_copy(b_h, b_v)

    lanes = lax.iota(jnp.int32, 16)

    def _perm(x, idx):
        return x.at[idx].get(mode="promise_in_bounds")

    def _allsum(x):
        for sh in (1, 2, 4, 8):
            x = x + _perm(x, lanes ^ sh)
        return x

    for j in range(D // 16):
        def fold_step(t, carry):
            accu, acci = carry
            k = j * 16 + t
            su = jnp.zeros((16,), jnp.float32)
            si = jnp.zeros((16,), jnp.float32)
            for m in range(LAT // 16):
                wc_u = w_v[pl.ds(m * 16, 16)]
                wc_i = w_v[pl.ds(LAT + m * 16, 16)]
                su = su + uA_v[k, pl.ds(m * 16, 16)] * wc_u
                si = si + iA_v[k, pl.ds(m * 16, 16)] * wc_i
            accu = jnp.where(lanes == t, _allsum(su), accu)
            acci = jnp.where(lanes == t, _allsum(si), acci)
            return accu, acci

        z = jnp.zeros((16,), jnp.float32)
        accu, acci = lax.fori_loop(0, 16, fold_step, (z, z))
        wu_v[pl.ds(j * 16, 16)] = accu
        wi_v[pl.ds(j * 16, 16)] = acci

    for cp in copies:
        cp.wait()

    bias = b_v[...]

    def group(g, _):
        rbase = pl.multiple_of(g * 16, 16)
        ob = jnp.zeros((16,), jnp.float32)
        for r in range(16):
            p = jnp.zeros((16,), jnp.float32)
            for m in range(D // 16):
                p = p + urows[rbase + r, pl.ds(m * 16, 16)] * wu_v[pl.ds(m * 16, 16)]
                p = p + irows[rbase + r, pl.ds(m * 16, 16)] * wi_v[pl.ds(m * 16, 16)]
            ob = jnp.where(lanes == r, _allsum(p), ob)
        out_v[pl.ds(rbase, 16)] = ob + bias
        return 0

    lax.fori_loop(0, BPW // 16, group, 0)
    pltpu.sync_copy(out_v, out_h.at[pl.ds(base, BPW)])


@functools.partial(
    pl.kernel,
    out_type=jax.ShapeDtypeStruct((B,), jnp.float32),
    mesh=plsc.VectorSubcoreMesh(core_axis_name="c", subcore_axis_name="s"),
    compiler_params=pltpu.CompilerParams(use_tc_tiling_on_sc=False),
    scratch_types=[
        pltpu.VMEM((NCH, ICH), jnp.int32),
        pltpu.VMEM((NCH, ICH), jnp.int32),
        pltpu.VMEM((BPW, D), jnp.float32),
        pltpu.VMEM((BPW, D), jnp.float32),
        pltpu.VMEM((D, LAT), jnp.float32),
        pltpu.VMEM((D, LAT), jnp.float32),
        pltpu.VMEM((2 * LAT,), jnp.float32),
        pltpu.VMEM((16,), jnp.float32),
        pltpu.VMEM((D,), jnp.float32),
        pltpu.VMEM((D,), jnp.float32),
        pltpu.VMEM((BPW,), jnp.float32),
        pltpu.SemaphoreType.DMA,
    ],
)
def _sc_ncf(uidx_h, iidx_h, uT_h, iT_h, uA_h, iA_h, w_h, b_h, out_h, *scratch):
    _body(uidx_h, iidx_h, uT_h, iT_h, uA_h, iA_h, w_h, b_h, out_h, *scratch)


def kernel(user_indices, item_indices, user_T, item_T, user_A, item_A,
           W_aff, b_aff):
    uidx = user_indices.astype(jnp.int32).reshape(NW, NCH, ICH)
    iidx = item_indices.astype(jnp.int32).reshape(NW, NCH, ICH)
    w = W_aff.reshape(2 * LAT)
    b16 = jnp.broadcast_to(b_aff, (16,))
    out = _sc_ncf(uidx, iidx, user_T, item_T, user_A, item_A, w, b16)
    return out.reshape(B, 1)


# speedup vs baseline: 1.0053x; 1.0053x over previous
import functools

import jax
import jax.numpy as jnp
from jax import lax
from jax.experimental import pallas as pl
from jax.experimental.pallas import tpu as pltpu
from jax.experimental.pallas import tpu_sc as plsc

B = 16384
D = 64
LAT = 128
NC = 2
NS = 16
NW = NC * NS
BPW = B // NW
ICH = 128
NCH = BPW // ICH


def _body(uidx_h, iidx_h, uT_h, iT_h, uA_h, iA_h, w_h, b_h, out_h,
          uidx_v, iidx_v, urows, irows, uA_v, iA_v, w_v, b_v,
          wu_v, wi_v, out_v, gsem):
    cid = lax.axis_index("c")
    sid = lax.axis_index("s")
    wid = sid * NC + cid
    base = wid * BPW

    pltpu.sync_copy(uidx_h.at[wid], uidx_v)
    pltpu.sync_copy(iidx_h.at[wid], iidx_v)
    copies = []
    for j in range(NCH):
        copies.append(pltpu.async_copy(
            uT_h.at[uidx_v.at[j]], urows.at[pl.ds(j * ICH, ICH)], gsem))
        copies.append(pltpu.async_copy(
            iT_h.at[iidx_v.at[j]], irows.at[pl.ds(j * ICH, ICH)], gsem))

    pltpu.sync_copy(uA_h, uA_v)
    pltpu.sync_copy(iA_h, iA_v)
    pltpu.sync_copy(w_h, w_v)
    pltpu.sync_copy(b_h, b_v)

    lanes = lax.iota(jnp.int32, 16)

    def _perm(x, idx):
        return x.at[idx].get(mode="promise_in_bounds")

    def _allsum(x):
        for sh in (1, 2, 4, 8):
            x = x + _perm(x, lanes ^ sh)
        return x

    scope_fold = jax.named_scope("ncf_fold")
    scope_fold.__enter__()
    for j in range(D // 16):
        def fold_step(t, carry):
            accu, acci = carry
            k = j * 16 + t
            su = jnp.zeros((16,), jnp.float32)
            si = jnp.zeros((16,), jnp.float32)
            for m in range(LAT // 16):
                wc_u = w_v[pl.ds(m * 16, 16)]
                wc_i = w_v[pl.ds(LAT + m * 16, 16)]
                su = su + uA_v[k, pl.ds(m * 16, 16)] * wc_u
                si = si + iA_v[k, pl.ds(m * 16, 16)] * wc_i
            accu = jnp.where(lanes == t, _allsum(su), accu)
            acci = jnp.where(lanes == t, _allsum(si), acci)
            return accu, acci

        z = jnp.zeros((16,), jnp.float32)
        accu, acci = lax.fori_loop(0, 16, fold_step, (z, z))
        wu_v[pl.ds(j * 16, 16)] = accu
        wi_v[pl.ds(j * 16, 16)] = acci

    scope_fold.__exit__(None, None, None)

    with jax.named_scope("ncf_drain"):
        for cp in copies:
            cp.wait()

    bias = b_v[...]

    def group(g, _):
        rbase = pl.multiple_of(g * 16, 16)
        ob = jnp.zeros((16,), jnp.float32)
        for r in range(16):
            p = jnp.zeros((16,), jnp.float32)
            for m in range(D // 16):
                p = p + urows[rbase + r, pl.ds(m * 16, 16)] * wu_v[pl.ds(m * 16, 16)]
                p = p + irows[rbase + r, pl.ds(m * 16, 16)] * wi_v[pl.ds(m * 16, 16)]
            ob = jnp.where(lanes == r, _allsum(p), ob)
        out_v[pl.ds(rbase, 16)] = ob + bias
        return 0

    with jax.named_scope("ncf_main"):
        lax.fori_loop(0, BPW // 16, group, 0)
    pltpu.sync_copy(out_v, out_h.at[pl.ds(base, BPW)])


@functools.partial(
    pl.kernel,
    out_type=jax.ShapeDtypeStruct((B,), jnp.float32),
    mesh=plsc.VectorSubcoreMesh(core_axis_name="c", subcore_axis_name="s"),
    compiler_params=pltpu.CompilerParams(use_tc_tiling_on_sc=False),
    scratch_types=[
        pltpu.VMEM((NCH, ICH), jnp.int32),
        pltpu.VMEM((NCH, ICH), jnp.int32),
        pltpu.VMEM((BPW, D), jnp.float32),
        pltpu.VMEM((BPW, D), jnp.float32),
        pltpu.VMEM((D, LAT), jnp.float32),
        pltpu.VMEM((D, LAT), jnp.float32),
        pltpu.VMEM((2 * LAT,), jnp.float32),
        pltpu.VMEM((16,), jnp.float32),
        pltpu.VMEM((D,), jnp.float32),
        pltpu.VMEM((D,), jnp.float32),
        pltpu.VMEM((BPW,), jnp.float32),
        pltpu.SemaphoreType.DMA,
    ],
)
def _sc_ncf(uidx_h, iidx_h, uT_h, iT_h, uA_h, iA_h, w_h, b_h, out_h, *scratch):
    _body(uidx_h, iidx_h, uT_h, iT_h, uA_h, iA_h, w_h, b_h, out_h, *scratch)


def kernel(user_indices, item_indices, user_T, item_T, user_A, item_A,
           W_aff, b_aff):
    uidx = user_indices.astype(jnp.int32).reshape(NW, NCH, ICH)
    iidx = item_indices.astype(jnp.int32).reshape(NW, NCH, ICH)
    w = W_aff.reshape(2 * LAT)
    b16 = jnp.broadcast_to(b_aff, (16,))
    out = _sc_ncf(uidx, iidx, user_T, item_T, user_A, item_A, w, b16)
    return out.reshape(B, 1)
